# baseline (speedup 1.0000x reference)
import functools

import jax
import jax.numpy as jnp
from jax import lax
from jax.experimental import pallas as pl
from jax.experimental.pallas import tpu as pltpu
from jax.experimental.pallas import tpu_sc as plsc

_VOCAB = 100000
_EMBED = 128
_BATCH = 4096
_SEQ = 200

_NC = 2
_NS = 16
_NW = _NC * _NS

_NCHUNK = 8
_CB = _BATCH // _NCHUNK
_IDX_CHUNK = _CB * _SEQ
_PER_W = _IDX_CHUNK // _NW
_ROWS = 128
_GATHERS = _PER_W // _ROWS

_BB = 16
_STEPS = _CB // _BB


def _sc_gather(x3d, W):
    mesh = plsc.VectorSubcoreMesh(core_axis_name="c", subcore_axis_name="s")

    @functools.partial(
        pl.kernel,
        mesh=mesh,
        out_type=jax.ShapeDtypeStruct((_IDX_CHUNK, _EMBED), jnp.float32),
        scratch_types=[
            pltpu.VMEM((_GATHERS, _ROWS), jnp.int32),
            pltpu.VMEM((2, _ROWS, _EMBED), jnp.float32),
            pltpu.SemaphoreType.DMA,
            pltpu.SemaphoreType.DMA,
        ],
    )
    def k(x_hbm, w_hbm, out_hbm, idx_v, rows_v, sem0, sem1):
        wid = lax.axis_index("s") * _NC + lax.axis_index("c")
        pltpu.sync_copy(x_hbm.at[wid], idx_v)
        base = wid * _PER_W

        def _out(j):
            return out_hbm.at[pl.ds(base + j * _ROWS, _ROWS)]

        pltpu.async_copy(w_hbm.at[idx_v.at[0]], rows_v.at[0], sem0)

        def body(i, carry):
            j0 = 2 * i
            pltpu.async_copy(w_hbm.at[idx_v.at[j0 + 1]], rows_v.at[1], sem1)
            pltpu.make_async_copy(
                w_hbm.at[idx_v.at[j0]], rows_v.at[0], sem0
            ).wait()
            pltpu.sync_copy(rows_v.at[0], _out(j0))
            pltpu.async_copy(w_hbm.at[idx_v.at[j0 + 2]], rows_v.at[0], sem0)
            pltpu.make_async_copy(
                w_hbm.at[idx_v.at[j0 + 1]], rows_v.at[1], sem1
            ).wait()
            pltpu.sync_copy(rows_v.at[1], _out(j0 + 1))
            return carry

        lax.fori_loop(0, (_GATHERS - 1) // 2, body, 0)
        pltpu.make_async_copy(
            w_hbm.at[idx_v.at[_GATHERS - 1]], rows_v.at[0], sem0
        ).wait()
        pltpu.sync_copy(rows_v.at[0], _out(_GATHERS - 1))

    return k(x3d, W)


def _tc_transpose_chunk(acc, G, kidx):
    if acc is None:
        def body0(g_ref, o_ref):
            o_ref[...] = jnp.swapaxes(g_ref[...], 1, 2)

        return pl.pallas_call(
            body0,
            grid=(_STEPS,),
            in_specs=[
                pl.BlockSpec((_BB, _SEQ, _EMBED), lambda i: (i, 0, 0)),
            ],
            out_specs=pl.BlockSpec(
                (_BB, _EMBED, _SEQ), lambda i, k=kidx: (k * _STEPS + i, 0, 0)
            ),
            out_shape=jax.ShapeDtypeStruct((_BATCH, _EMBED, _SEQ), jnp.float32),
        )(G)

    def body(_, g_ref, o_ref):
        o_ref[...] = jnp.swapaxes(g_ref[...], 1, 2)

    return pl.pallas_call(
        body,
        grid=(_STEPS,),
        in_specs=[
            pl.BlockSpec(memory_space=pl.ANY),
            pl.BlockSpec((_BB, _SEQ, _EMBED), lambda i: (i, 0, 0)),
        ],
        out_specs=pl.BlockSpec(
            (_BB, _EMBED, _SEQ), lambda i, k=kidx: (k * _STEPS + i, 0, 0)
        ),
        out_shape=jax.ShapeDtypeStruct((_BATCH, _EMBED, _SEQ), jnp.float32),
        input_output_aliases={0: 0},
    )(acc, G)


def kernel(x, W):
    x3 = x.reshape(_NCHUNK, _NW, _GATHERS, _ROWS)
    acc = None
    for k in range(_NCHUNK):
        G = _sc_gather(x3[k], W)
        acc = _tc_transpose_chunk(acc, G.reshape(_CB, _SEQ, _EMBED), k)
    return acc


# speedup vs baseline: 1.0262x; 1.0262x over previous
import functools

import jax
import jax.numpy as jnp
from jax import lax
from jax.experimental import pallas as pl
from jax.experimental.pallas import tpu as pltpu
from jax.experimental.pallas import tpu_sc as plsc

_VOCAB = 100000
_EMBED = 128
_BATCH = 4096
_SEQ = 200

_NC = 2
_NS = 16
_NW = _NC * _NS

_NCHUNK = 8
_CB = _BATCH // _NCHUNK
_IDX_CHUNK = _CB * _SEQ
_PER_W = _IDX_CHUNK // _NW
_ROWS = 128
_GATHERS = _PER_W // _ROWS

_BB = 32
_STEPS = _CB // _BB


def _sc_gather(x3d, W):
    mesh = plsc.VectorSubcoreMesh(core_axis_name="c", subcore_axis_name="s")

    @functools.partial(
        pl.kernel,
        mesh=mesh,
        out_type=jax.ShapeDtypeStruct((_IDX_CHUNK, _EMBED), jnp.float32),
        scratch_types=[
            pltpu.VMEM((_GATHERS, _ROWS), jnp.int32),
            pltpu.VMEM((2, _ROWS, _EMBED), jnp.float32),
            pltpu.SemaphoreType.DMA,
            pltpu.SemaphoreType.DMA,
        ],
    )
    def k(x_hbm, w_hbm, out_hbm, idx_v, rows_v, sem0, sem1):
        wid = lax.axis_index("s") * _NC + lax.axis_index("c")
        pltpu.sync_copy(x_hbm.at[wid], idx_v)
        base = wid * _PER_W

        def _out(j):
            return out_hbm.at[pl.ds(base + j * _ROWS, _ROWS)]

        pltpu.async_copy(w_hbm.at[idx_v.at[0]], rows_v.at[0], sem0)

        def body(i, carry):
            j0 = 2 * i
            pltpu.async_copy(w_hbm.at[idx_v.at[j0 + 1]], rows_v.at[1], sem1)
            pltpu.make_async_copy(
                w_hbm.at[idx_v.at[j0]], rows_v.at[0], sem0
            ).wait()
            pltpu.sync_copy(rows_v.at[0], _out(j0))
            pltpu.async_copy(w_hbm.at[idx_v.at[j0 + 2]], rows_v.at[0], sem0)
            pltpu.make_async_copy(
                w_hbm.at[idx_v.at[j0 + 1]], rows_v.at[1], sem1
            ).wait()
            pltpu.sync_copy(rows_v.at[1], _out(j0 + 1))
            return carry

        lax.fori_loop(0, (_GATHERS - 1) // 2, body, 0)
        pltpu.make_async_copy(
            w_hbm.at[idx_v.at[_GATHERS - 1]], rows_v.at[0], sem0
        ).wait()
        pltpu.sync_copy(rows_v.at[0], _out(_GATHERS - 1))

    return k(x3d, W)


def _tc_transpose_chunk(acc, G, kidx):
    if acc is None:
        def body0(g_ref, o_ref):
            o_ref[...] = jnp.swapaxes(g_ref[...], 1, 2)

        return pl.pallas_call(
            body0,
            grid=(_STEPS,),
            in_specs=[
                pl.BlockSpec((_BB, _SEQ, _EMBED), lambda i: (i, 0, 0)),
            ],
            out_specs=pl.BlockSpec(
                (_BB, _EMBED, _SEQ), lambda i, k=kidx: (k * _STEPS + i, 0, 0)
            ),
            out_shape=jax.ShapeDtypeStruct((_BATCH, _EMBED, _SEQ), jnp.float32),
        )(G)

    def body(_, g_ref, o_ref):
        o_ref[...] = jnp.swapaxes(g_ref[...], 1, 2)

    return pl.pallas_call(
        body,
        grid=(_STEPS,),
        in_specs=[
            pl.BlockSpec(memory_space=pl.ANY),
            pl.BlockSpec((_BB, _SEQ, _EMBED), lambda i: (i, 0, 0)),
        ],
        out_specs=pl.BlockSpec(
            (_BB, _EMBED, _SEQ), lambda i, k=kidx: (k * _STEPS + i, 0, 0)
        ),
        out_shape=jax.ShapeDtypeStruct((_BATCH, _EMBED, _SEQ), jnp.float32),
        input_output_aliases={0: 0},
    )(acc, G)


def kernel(x, W):
    x3 = x.reshape(_NCHUNK, _NW, _GATHERS, _ROWS)
    acc = None
    for k in range(_NCHUNK):
        G = _sc_gather(x3[k], W)
        acc = _tc_transpose_chunk(acc, G.reshape(_CB, _SEQ, _EMBED), k)
    return acc


# speedup vs baseline: 1.0347x; 1.0083x over previous
import functools

import jax
import jax.numpy as jnp
from jax import lax
from jax.experimental import pallas as pl
from jax.experimental.pallas import tpu as pltpu
from jax.experimental.pallas import tpu_sc as plsc

_VOCAB = 100000
_EMBED = 128
_BATCH = 4096
_SEQ = 200

_NC = 2
_NS = 16
_NW = _NC * _NS

_NCHUNK = 8
_CB = _BATCH // _NCHUNK
_IDX_CHUNK = _CB * _SEQ
_PER_W = _IDX_CHUNK // _NW
_ROWS = 128
_GATHERS = _PER_W // _ROWS

_BB = 64
_STEPS = _CB // _BB


def _sc_gather(x3d, W):
    mesh = plsc.VectorSubcoreMesh(core_axis_name="c", subcore_axis_name="s")

    @functools.partial(
        pl.kernel,
        mesh=mesh,
        out_type=jax.ShapeDtypeStruct((_IDX_CHUNK, _EMBED), jnp.float32),
        scratch_types=[
            pltpu.VMEM((_GATHERS, _ROWS), jnp.int32),
            pltpu.VMEM((2, _ROWS, _EMBED), jnp.float32),
            pltpu.SemaphoreType.DMA,
            pltpu.SemaphoreType.DMA,
        ],
    )
    def k(x_hbm, w_hbm, out_hbm, idx_v, rows_v, sem0, sem1):
        wid = lax.axis_index("s") * _NC + lax.axis_index("c")
        pltpu.sync_copy(x_hbm.at[wid], idx_v)
        base = wid * _PER_W

        def _out(j):
            return out_hbm.at[pl.ds(base + j * _ROWS, _ROWS)]

        pltpu.async_copy(w_hbm.at[idx_v.at[0]], rows_v.at[0], sem0)

        def body(i, carry):
            j0 = 2 * i
            pltpu.async_copy(w_hbm.at[idx_v.at[j0 + 1]], rows_v.at[1], sem1)
            pltpu.make_async_copy(
                w_hbm.at[idx_v.at[j0]], rows_v.at[0], sem0
            ).wait()
            pltpu.sync_copy(rows_v.at[0], _out(j0))
            pltpu.async_copy(w_hbm.at[idx_v.at[j0 + 2]], rows_v.at[0], sem0)
            pltpu.make_async_copy(
                w_hbm.at[idx_v.at[j0 + 1]], rows_v.at[1], sem1
            ).wait()
            pltpu.sync_copy(rows_v.at[1], _out(j0 + 1))
            return carry

        lax.fori_loop(0, (_GATHERS - 1) // 2, body, 0)
        pltpu.make_async_copy(
            w_hbm.at[idx_v.at[_GATHERS - 1]], rows_v.at[0], sem0
        ).wait()
        pltpu.sync_copy(rows_v.at[0], _out(_GATHERS - 1))

    return k(x3d, W)


def _tc_transpose_chunk(acc, G, kidx):
    if acc is None:
        def body0(g_ref, o_ref):
            o_ref[...] = jnp.swapaxes(g_ref[...], 1, 2)

        return pl.pallas_call(
            body0,
            grid=(_STEPS,),
            in_specs=[
                pl.BlockSpec((_BB, _SEQ, _EMBED), lambda i: (i, 0, 0)),
            ],
            out_specs=pl.BlockSpec(
                (_BB, _EMBED, _SEQ), lambda i, k=kidx: (k * _STEPS + i, 0, 0)
            ),
            out_shape=jax.ShapeDtypeStruct((_BATCH, _EMBED, _SEQ), jnp.float32),
        )(G)

    def body(_, g_ref, o_ref):
        o_ref[...] = jnp.swapaxes(g_ref[...], 1, 2)

    return pl.pallas_call(
        body,
        grid=(_STEPS,),
        in_specs=[
            pl.BlockSpec(memory_space=pl.ANY),
            pl.BlockSpec((_BB, _SEQ, _EMBED), lambda i: (i, 0, 0)),
        ],
        out_specs=pl.BlockSpec(
            (_BB, _EMBED, _SEQ), lambda i, k=kidx: (k * _STEPS + i, 0, 0)
        ),
        out_shape=jax.ShapeDtypeStruct((_BATCH, _EMBED, _SEQ), jnp.float32),
        input_output_aliases={0: 0},
    )(acc, G)


def kernel(x, W):
    x3 = x.reshape(_NCHUNK, _NW, _GATHERS, _ROWS)
    acc = None
    for k in range(_NCHUNK):
        G = _sc_gather(x3[k], W)
        acc = _tc_transpose_chunk(acc, G.reshape(_CB, _SEQ, _EMBED), k)
    return acc
